# baseline (speedup 1.0000x reference)
import functools

import jax
import jax.numpy as jnp
from jax import lax
from jax.experimental import pallas as pl
from jax.experimental.pallas import tpu as pltpu
from jax.experimental.pallas import tpu_sc as plsc

B = 16384
D = 64
NPIX = 65536
NFRM = 10000
NW = 32
FPW = D // NW
NGRP = B // 16

_W1A, _W1B, _W1C = 0, 64, 128
_WLEN = 192


def _sigmoid(z):
    return 1.0 / (1.0 + jnp.exp(-z))


def _vslice(vt, d):
    return vt.at[pl.ds(pl.multiple_of(d * NFRM, 8), NFRM)]


def _sc_body(pixel, frame, uf, vf, up1f, up2f, vp1f, vp2f, wpack,
             out,
             pix_v, frm_v, acc, urow, vrow, wv, semu, semv):
    wid = lax.axis_index("s") * 2 + lax.axis_index("c")

    d0 = wid * FPW
    cu = pltpu.async_copy(uf.at[d0 >> 3, :, d0 & 7, :], urow, semu)
    cv = pltpu.async_copy(_vslice(vp1f, d0), vrow, semv)
    pltpu.sync_copy(wpack, wv)
    pltpu.sync_copy(pixel, pix_v)
    pltpu.sync_copy(frame, frm_v)

    for j in range(FPW):
        d = wid * FPW + j
        fb = d >> 3
        f = d & 7
        wad = plsc.load_gather(wv, [jnp.full((16,), _W1A, jnp.int32) + d])
        wbd = plsc.load_gather(wv, [jnp.full((16,), _W1B, jnp.int32) + d])
        wcd = plsc.load_gather(wv, [jnp.full((16,), _W1C, jnp.int32) + d])

        cu.wait()
        if j == 0:
            @plsc.parallel_loop(0, NGRP, 1, unroll=4)
            def u_loop(g):
                gof = pl.multiple_of(g * 16, 16)
                pix = pix_v[pl.ds(gof, 16)]
                val = plsc.load_gather(urow, [pix >> 7, pix & 127])
                acc[pl.ds(gof, 16)] = val * wad
        else:
            @plsc.parallel_loop(0, NGRP, 1, unroll=4)
            def u_loop(g):
                gof = pl.multiple_of(g * 16, 16)
                pix = pix_v[pl.ds(gof, 16)]
                val = plsc.load_gather(urow, [pix >> 7, pix & 127])
                acc[pl.ds(gof, 16)] = acc[pl.ds(gof, 16)] + val * wad
        cu = pltpu.async_copy(up1f.at[fb, :, f, :], urow, semu)

        cu.wait()
        cv.wait()

        @plsc.parallel_loop(0, NGRP, 1, unroll=4)
        def p1_loop(g):
            gof = pl.multiple_of(g * 16, 16)
            pix = pix_v[pl.ds(gof, 16)]
            a = plsc.load_gather(urow, [pix >> 7, pix & 127])
            b = plsc.load_gather(vrow, [frm_v[pl.ds(gof, 16)]])
            t = jnp.maximum(a, 0.0) * jnp.maximum(b, 0.0)
            acc[pl.ds(gof, 16)] = acc[pl.ds(gof, 16)] + t * wcd
        cu = pltpu.async_copy(up2f.at[fb, :, f, :], urow, semu)
        cv = pltpu.async_copy(_vslice(vf, d), vrow, semv)

        cv.wait()

        @plsc.parallel_loop(0, NGRP, 1, unroll=4)
        def v_loop(g):
            gof = pl.multiple_of(g * 16, 16)
            val = plsc.load_gather(vrow, [frm_v[pl.ds(gof, 16)]])
            acc[pl.ds(gof, 16)] = acc[pl.ds(gof, 16)] + val * wbd
        cv = pltpu.async_copy(_vslice(vp2f, d), vrow, semv)

        cu.wait()
        cv.wait()

        @plsc.parallel_loop(0, NGRP, 1, unroll=4)
        def p2_loop(g):
            gof = pl.multiple_of(g * 16, 16)
            pix = pix_v[pl.ds(gof, 16)]
            a = plsc.load_gather(urow, [pix >> 7, pix & 127])
            b = plsc.load_gather(vrow, [frm_v[pl.ds(gof, 16)]])
            t = jnp.maximum(a, 0.0) * jnp.maximum(b, 0.0)
            acc[pl.ds(gof, 16)] = acc[pl.ds(gof, 16)] + t * wcd

        if j + 1 < FPW:
            dn = d + 1
            cu = pltpu.async_copy(uf.at[dn >> 3, :, dn & 7, :], urow, semu)
            cv = pltpu.async_copy(_vslice(vp1f, dn), vrow, semv)

    pltpu.sync_copy(acc, out.at[wid])


@jax.jit
def _sc_run(pixel, frame, uf, vf, up1f, up2f, vp1f, vp2f, wpack):
    mesh = plsc.VectorSubcoreMesh(core_axis_name="c", subcore_axis_name="s",
                                  num_cores=2, num_subcores=16)
    f = functools.partial(
        pl.kernel,
        out_type=jax.ShapeDtypeStruct((NW, B), jnp.float32),
        mesh=mesh,
        compiler_params=pltpu.CompilerParams(needs_layout_passes=False,
                                             use_tc_tiling_on_sc=False),
        scratch_types=[
            pltpu.VMEM((B,), jnp.int32),
            pltpu.VMEM((B,), jnp.int32),
            pltpu.VMEM((B,), jnp.float32),
            pltpu.VMEM((NPIX // 128, 128), jnp.float32),
            pltpu.VMEM((NFRM,), jnp.float32),
            pltpu.VMEM((_WLEN,), jnp.float32),
            pltpu.SemaphoreType.DMA,
            pltpu.SemaphoreType.DMA,
        ],
    )(_sc_body)
    return f(pixel, frame, uf, vf, up1f, up2f, vp1f, vp2f, wpack)


def _tc_body(p_ref, t_ref, wsc, s1, bs1, s2, bs2, s3, bs3, s4, bs4,
             x_ref, s_ref):
    h = jnp.sum(p_ref[...], axis=0) + wsc[0]
    h = jnp.maximum(h, 0.0)
    x = _sigmoid(h * wsc[1] + wsc[2])
    x_ref[...] = x
    s = t_ref[...] - x
    a1 = [jnp.maximum(s * s1[0, j] + bs1[j], 0.0) for j in range(10)]
    a2 = []
    for j in range(10):
        acc = jnp.full_like(s, bs2[j])
        for k in range(10):
            acc = acc + a1[k] * s2[k, j]
        a2.append(jnp.maximum(acc, 0.0))
    a3 = []
    for j in range(10):
        acc = jnp.full_like(s, bs3[j])
        for k in range(10):
            acc = acc + a2[k] * s3[k, j]
        a3.append(jnp.maximum(acc, 0.0))
    z = jnp.full_like(s, bs4[0])
    for k in range(10):
        z = z + a3[k] * s4[k, 0]
    s_ref[...] = _sigmoid(z)


@jax.jit
def _tc_run(partial, target, wsc, S1, bs1, S2, bs2, S3, bs3, S4, bs4):
    smem = lambda: pl.BlockSpec(memory_space=pltpu.SMEM)
    vmem = lambda: pl.BlockSpec(memory_space=pltpu.VMEM)
    return pl.pallas_call(
        _tc_body,
        in_specs=[
            vmem(), vmem(), smem(), smem(), smem(), smem(), smem(),
            smem(), smem(), smem(), smem(),
        ],
        out_specs=(vmem(), vmem()),
        out_shape=(jax.ShapeDtypeStruct((8, B // 8), jnp.float32),
                   jax.ShapeDtypeStruct((8, B // 8), jnp.float32)),
    )(partial, target, wsc, S1, bs1, S2, bs2, S3, bs3, S4, bs4)


def kernel(pixel, frame, target, U, V, Up1, Up2, Vp1, Vp2, W1, b1, W2, b2,
           S1, bs1, S2, bs2, S3, bs3, S4, bs4):
    wpack = jnp.concatenate([W1[:, 0]])
    wsc = jnp.concatenate([b1, W2[0], b2])
    zview = lambda t: t.T.reshape(8, 8, NPIX // 128, 128).transpose(0, 2, 1, 3)
    partial = _sc_run(pixel.astype(jnp.int32), frame.astype(jnp.int32),
                      zview(U), V.T.reshape(-1),
                      zview(Up1), zview(Up2),
                      Vp1.T.reshape(-1), Vp2.T.reshape(-1), wpack)
    x, s = _tc_run(partial.reshape(NW, 8, B // 8), target.reshape(8, B // 8),
                   wsc, S1, bs1, S2, bs2, S3, bs3, S4, bs4)
    return (x.reshape(B, 1), s.reshape(B, 1))


# speedup vs baseline: 1.0094x; 1.0094x over previous
import functools

import jax
import jax.numpy as jnp
from jax import lax
from jax.experimental import pallas as pl
from jax.experimental.pallas import tpu as pltpu
from jax.experimental.pallas import tpu_sc as plsc

B = 16384
D = 64
NPIX = 65536
NFRM = 10000
NW = 32
FPW = D // NW
NGRP = B // 16

_W1A, _W1B, _W1C = 0, 64, 128
_WLEN = 192


def _sigmoid(z):
    return 1.0 / (1.0 + jnp.exp(-z))


def _vslice(vt, d):
    return vt.at[pl.ds(pl.multiple_of(d * NFRM, 8), NFRM)]


def _sc_body(pixel, frame, uf, vf, up1f, up2f, vp1f, vp2f, wpack,
             out,
             pix_v, frm_v, acc, urow, vrow, wv, semu, semv):
    wid = lax.axis_index("s") * 2 + lax.axis_index("c")

    d0 = wid * FPW
    cu = pltpu.async_copy(uf.at[d0 >> 3, :, d0 & 7, :], urow, semu)
    cv = pltpu.async_copy(_vslice(vp1f, d0), vrow, semv)
    pltpu.sync_copy(wpack, wv)
    pltpu.sync_copy(pixel, pix_v)
    pltpu.sync_copy(frame, frm_v)

    for j in range(FPW):
        d = wid * FPW + j
        fb = d >> 3
        f = d & 7
        wad = plsc.load_gather(wv, [jnp.full((16,), _W1A, jnp.int32) + d])
        wbd = plsc.load_gather(wv, [jnp.full((16,), _W1B, jnp.int32) + d])
        wcd = plsc.load_gather(wv, [jnp.full((16,), _W1C, jnp.int32) + d])

        cu.wait()
        if j == 0:
            @plsc.parallel_loop(0, NGRP, 1, unroll=8)
            def u_loop(g):
                gof = pl.multiple_of(g * 16, 16)
                pix = pix_v[pl.ds(gof, 16)]
                val = plsc.load_gather(urow, [pix >> 7, pix & 127])
                acc[pl.ds(gof, 16)] = val * wad
        else:
            @plsc.parallel_loop(0, NGRP, 1, unroll=8)
            def u_loop(g):
                gof = pl.multiple_of(g * 16, 16)
                pix = pix_v[pl.ds(gof, 16)]
                val = plsc.load_gather(urow, [pix >> 7, pix & 127])
                acc[pl.ds(gof, 16)] = acc[pl.ds(gof, 16)] + val * wad
        cu = pltpu.async_copy(up1f.at[fb, :, f, :], urow, semu)

        cu.wait()
        cv.wait()

        @plsc.parallel_loop(0, NGRP, 1, unroll=8)
        def p1_loop(g):
            gof = pl.multiple_of(g * 16, 16)
            pix = pix_v[pl.ds(gof, 16)]
            a = plsc.load_gather(urow, [pix >> 7, pix & 127])
            b = plsc.load_gather(vrow, [frm_v[pl.ds(gof, 16)]])
            t = jnp.maximum(a, 0.0) * jnp.maximum(b, 0.0)
            acc[pl.ds(gof, 16)] = acc[pl.ds(gof, 16)] + t * wcd
        cu = pltpu.async_copy(up2f.at[fb, :, f, :], urow, semu)
        cv = pltpu.async_copy(_vslice(vf, d), vrow, semv)

        cv.wait()

        @plsc.parallel_loop(0, NGRP, 1, unroll=8)
        def v_loop(g):
            gof = pl.multiple_of(g * 16, 16)
            val = plsc.load_gather(vrow, [frm_v[pl.ds(gof, 16)]])
            acc[pl.ds(gof, 16)] = acc[pl.ds(gof, 16)] + val * wbd
        cv = pltpu.async_copy(_vslice(vp2f, d), vrow, semv)

        cu.wait()
        cv.wait()

        @plsc.parallel_loop(0, NGRP, 1, unroll=8)
        def p2_loop(g):
            gof = pl.multiple_of(g * 16, 16)
            pix = pix_v[pl.ds(gof, 16)]
            a = plsc.load_gather(urow, [pix >> 7, pix & 127])
            b = plsc.load_gather(vrow, [frm_v[pl.ds(gof, 16)]])
            t = jnp.maximum(a, 0.0) * jnp.maximum(b, 0.0)
            acc[pl.ds(gof, 16)] = acc[pl.ds(gof, 16)] + t * wcd

        if j + 1 < FPW:
            dn = d + 1
            cu = pltpu.async_copy(uf.at[dn >> 3, :, dn & 7, :], urow, semu)
            cv = pltpu.async_copy(_vslice(vp1f, dn), vrow, semv)

    pltpu.sync_copy(acc, out.at[wid])


@jax.jit
def _sc_run(pixel, frame, uf, vf, up1f, up2f, vp1f, vp2f, wpack):
    mesh = plsc.VectorSubcoreMesh(core_axis_name="c", subcore_axis_name="s",
                                  num_cores=2, num_subcores=16)
    f = functools.partial(
        pl.kernel,
        out_type=jax.ShapeDtypeStruct((NW, B), jnp.float32),
        mesh=mesh,
        compiler_params=pltpu.CompilerParams(needs_layout_passes=False,
                                             use_tc_tiling_on_sc=False),
        scratch_types=[
            pltpu.VMEM((B,), jnp.int32),
            pltpu.VMEM((B,), jnp.int32),
            pltpu.VMEM((B,), jnp.float32),
            pltpu.VMEM((NPIX // 128, 128), jnp.float32),
            pltpu.VMEM((NFRM,), jnp.float32),
            pltpu.VMEM((_WLEN,), jnp.float32),
            pltpu.SemaphoreType.DMA,
            pltpu.SemaphoreType.DMA,
        ],
    )(_sc_body)
    return f(pixel, frame, uf, vf, up1f, up2f, vp1f, vp2f, wpack)


def _tc_body(p_ref, t_ref, wsc, s1, bs1, s2, bs2, s3, bs3, s4, bs4,
             x_ref, s_ref):
    h = jnp.sum(p_ref[...], axis=0) + wsc[0]
    h = jnp.maximum(h, 0.0)
    x = _sigmoid(h * wsc[1] + wsc[2])
    x_ref[...] = x
    s = t_ref[...] - x
    a1 = [jnp.maximum(s * s1[0, j] + bs1[j], 0.0) for j in range(10)]
    a2 = []
    for j in range(10):
        acc = jnp.full_like(s, bs2[j])
        for k in range(10):
            acc = acc + a1[k] * s2[k, j]
        a2.append(jnp.maximum(acc, 0.0))
    a3 = []
    for j in range(10):
        acc = jnp.full_like(s, bs3[j])
        for k in range(10):
            acc = acc + a2[k] * s3[k, j]
        a3.append(jnp.maximum(acc, 0.0))
    z = jnp.full_like(s, bs4[0])
    for k in range(10):
        z = z + a3[k] * s4[k, 0]
    s_ref[...] = _sigmoid(z)


@jax.jit
def _tc_run(partial, target, wsc, S1, bs1, S2, bs2, S3, bs3, S4, bs4):
    smem = lambda: pl.BlockSpec(memory_space=pltpu.SMEM)
    vmem = lambda: pl.BlockSpec(memory_space=pltpu.VMEM)
    return pl.pallas_call(
        _tc_body,
        in_specs=[
            vmem(), vmem(), smem(), smem(), smem(), smem(), smem(),
            smem(), smem(), smem(), smem(),
        ],
        out_specs=(vmem(), vmem()),
        out_shape=(jax.ShapeDtypeStruct((8, B // 8), jnp.float32),
                   jax.ShapeDtypeStruct((8, B // 8), jnp.float32)),
    )(partial, target, wsc, S1, bs1, S2, bs2, S3, bs3, S4, bs4)


def kernel(pixel, frame, target, U, V, Up1, Up2, Vp1, Vp2, W1, b1, W2, b2,
           S1, bs1, S2, bs2, S3, bs3, S4, bs4):
    wpack = jnp.concatenate([W1[:, 0]])
    wsc = jnp.concatenate([b1, W2[0], b2])
    zview = lambda t: t.T.reshape(8, 8, NPIX // 128, 128).transpose(0, 2, 1, 3)
    partial = _sc_run(pixel.astype(jnp.int32), frame.astype(jnp.int32),
                      zview(U), V.T.reshape(-1),
                      zview(Up1), zview(Up2),
                      Vp1.T.reshape(-1), Vp2.T.reshape(-1), wpack)
    x, s = _tc_run(partial.reshape(NW, 8, B // 8), target.reshape(8, B // 8),
                   wsc, S1, bs1, S2, bs2, S3, bs3, S4, bs4)
    return (x.reshape(B, 1), s.reshape(B, 1))


# speedup vs baseline: 1.0235x; 1.0139x over previous
import functools

import jax
import jax.numpy as jnp
from jax import lax
from jax.experimental import pallas as pl
from jax.experimental.pallas import tpu as pltpu
from jax.experimental.pallas import tpu_sc as plsc

B = 16384
D = 64
NPIX = 65536
NFRM = 10000
NW = 32
FPW = D // NW
NGRP = B // 16

_W1A, _W1B, _W1C = 0, 64, 128
_WLEN = 192


def _sigmoid(z):
    return 1.0 / (1.0 + jnp.exp(-z))


def _vslice(vt, d):
    return vt.at[pl.ds(pl.multiple_of(d * NFRM, 8), NFRM)]


def _sc_body(pixel, frame, uf, vf, up1f, up2f, vp1f, vp2f, wpack,
             out,
             pix_v, frm_v, acc, urow, vrow, wv, semu, semv):
    wid = lax.axis_index("s") * 2 + lax.axis_index("c")

    d0 = wid * FPW
    cu = pltpu.async_copy(uf.at[d0 >> 3, :, d0 & 7, :], urow, semu)
    cv = pltpu.async_copy(_vslice(vf, d0), vrow, semv)
    pltpu.sync_copy(wpack, wv)
    pltpu.sync_copy(pixel, pix_v)
    pltpu.sync_copy(frame, frm_v)

    for j in range(FPW):
        d = wid * FPW + j
        fb = d >> 3
        f = d & 7
        wad = plsc.load_gather(wv, [jnp.full((16,), _W1A, jnp.int32) + d])
        wbd = plsc.load_gather(wv, [jnp.full((16,), _W1B, jnp.int32) + d])
        wcd = plsc.load_gather(wv, [jnp.full((16,), _W1C, jnp.int32) + d])

        cu.wait()
        if j == 0:
            @plsc.parallel_loop(0, NGRP, 1, unroll=4)
            def u_loop(g):
                gof = pl.multiple_of(g * 16, 16)
                pix = pix_v[pl.ds(gof, 16)]
                val = plsc.load_gather(urow, [pix >> 7, pix & 127])
                acc[pl.ds(gof, 16)] = val * wad
        else:
            @plsc.parallel_loop(0, NGRP, 1, unroll=4)
            def u_loop(g):
                gof = pl.multiple_of(g * 16, 16)
                pix = pix_v[pl.ds(gof, 16)]
                val = plsc.load_gather(urow, [pix >> 7, pix & 127])
                acc[pl.ds(gof, 16)] = acc[pl.ds(gof, 16)] + val * wad

        cu = pltpu.async_copy(up1f.at[fb, :, f, :], urow, semu)

        cv.wait()

        @plsc.parallel_loop(0, NGRP, 1, unroll=4)
        def v_loop(g):
            gof = pl.multiple_of(g * 16, 16)
            val = plsc.load_gather(vrow, [frm_v[pl.ds(gof, 16)]])
            acc[pl.ds(gof, 16)] = acc[pl.ds(gof, 16)] + val * wbd
        cv = pltpu.async_copy(_vslice(vp1f, d), vrow, semv)

        for step in range(2):
            cu.wait()
            cv.wait()

            @plsc.parallel_loop(0, NGRP, 1, unroll=4)
            def p_loop(g):
                gof = pl.multiple_of(g * 16, 16)
                pix = pix_v[pl.ds(gof, 16)]
                a = plsc.load_gather(urow, [pix >> 7, pix & 127])
                b = plsc.load_gather(vrow, [frm_v[pl.ds(gof, 16)]])
                t = jnp.maximum(a, 0.0) * jnp.maximum(b, 0.0)
                acc[pl.ds(gof, 16)] = acc[pl.ds(gof, 16)] + t * wcd

            if step == 0:
                cu = pltpu.async_copy(up2f.at[fb, :, f, :], urow, semu)
                cv = pltpu.async_copy(_vslice(vp2f, d), vrow, semv)
            elif j + 1 < FPW:
                dn = d + 1
                cu = pltpu.async_copy(uf.at[dn >> 3, :, dn & 7, :], urow,
                                      semu)
                cv = pltpu.async_copy(_vslice(vf, dn), vrow, semv)

    pltpu.sync_copy(acc, out.at[wid])


@jax.jit
def _sc_run(pixel, frame, uf, vf, up1f, up2f, vp1f, vp2f, wpack):
    mesh = plsc.VectorSubcoreMesh(core_axis_name="c", subcore_axis_name="s",
                                  num_cores=2, num_subcores=16)
    f = functools.partial(
        pl.kernel,
        out_type=jax.ShapeDtypeStruct((NW, B), jnp.float32),
        mesh=mesh,
        compiler_params=pltpu.CompilerParams(needs_layout_passes=False,
                                             use_tc_tiling_on_sc=False),
        scratch_types=[
            pltpu.VMEM((B,), jnp.int32),
            pltpu.VMEM((B,), jnp.int32),
            pltpu.VMEM((B,), jnp.float32),
            pltpu.VMEM((NPIX // 128, 128), jnp.float32),
            pltpu.VMEM((NFRM,), jnp.float32),
            pltpu.VMEM((_WLEN,), jnp.float32),
            pltpu.SemaphoreType.DMA,
            pltpu.SemaphoreType.DMA,
        ],
    )(_sc_body)
    return f(pixel, frame, uf, vf, up1f, up2f, vp1f, vp2f, wpack)


def _tc_body(p_ref, t_ref, wsc, s1, bs1, s2, bs2, s3, bs3, s4, bs4,
             x_ref, s_ref):
    h = jnp.sum(p_ref[...], axis=0) + wsc[0]
    h = jnp.maximum(h, 0.0)
    x = _sigmoid(h * wsc[1] + wsc[2])
    x_ref[...] = x
    s = t_ref[...] - x
    a1 = [jnp.maximum(s * s1[0, j] + bs1[j], 0.0) for j in range(10)]
    a2 = []
    for j in range(10):
        acc = jnp.full_like(s, bs2[j])
        for k in range(10):
            acc = acc + a1[k] * s2[k, j]
        a2.append(jnp.maximum(acc, 0.0))
    a3 = []
    for j in range(10):
        acc = jnp.full_like(s, bs3[j])
        for k in range(10):
            acc = acc + a2[k] * s3[k, j]
        a3.append(jnp.maximum(acc, 0.0))
    z = jnp.full_like(s, bs4[0])
    for k in range(10):
        z = z + a3[k] * s4[k, 0]
    s_ref[...] = _sigmoid(z)


@jax.jit
def _tc_run(partial, target, wsc, S1, bs1, S2, bs2, S3, bs3, S4, bs4):
    smem = lambda: pl.BlockSpec(memory_space=pltpu.SMEM)
    vmem = lambda: pl.BlockSpec(memory_space=pltpu.VMEM)
    return pl.pallas_call(
        _tc_body,
        in_specs=[
            vmem(), vmem(), smem(), smem(), smem(), smem(), smem(),
            smem(), smem(), smem(), smem(),
        ],
        out_specs=(vmem(), vmem()),
        out_shape=(jax.ShapeDtypeStruct((8, B // 8), jnp.float32),
                   jax.ShapeDtypeStruct((8, B // 8), jnp.float32)),
    )(partial, target, wsc, S1, bs1, S2, bs2, S3, bs3, S4, bs4)


def kernel(pixel, frame, target, U, V, Up1, Up2, Vp1, Vp2, W1, b1, W2, b2,
           S1, bs1, S2, bs2, S3, bs3, S4, bs4):
    wpack = jnp.concatenate([W1[:, 0]])
    wsc = jnp.concatenate([b1, W2[0], b2])
    zview = lambda t: t.T.reshape(8, 8, NPIX // 128, 128).transpose(0, 2, 1, 3)
    partial = _sc_run(pixel.astype(jnp.int32), frame.astype(jnp.int32),
                      zview(U), V.T.reshape(-1),
                      zview(Up1), zview(Up2),
                      Vp1.T.reshape(-1), Vp2.T.reshape(-1), wpack)
    x, s = _tc_run(partial.reshape(NW, 8, B // 8), target.reshape(8, B // 8),
                   wsc, S1, bs1, S2, bs2, S3, bs3, S4, bs4)
    return (x.reshape(B, 1), s.reshape(B, 1))
